# baseline (speedup 1.0000x reference)
import functools

import jax
import jax.numpy as jnp
from jax import lax
from jax.experimental import pallas as pl
from jax.experimental.pallas import tpu as pltpu
from jax.experimental.pallas import tpu_sc as plsc

N_NODES = 10000
OUT_DIM = 64
NPAD = 10240
E = 320000
NC = 1
NS = 16
L = 16
TILE_N = NPAD // NS
E1 = E // NS
E3 = E // (NC * NS)


def _fast_rsqrt(d):
    bi = plsc.bitcast(d, jnp.int32)
    y = plsc.bitcast(jnp.int32(0x5F3759DF) - (bi >> 1), jnp.float32)
    for _ in range(3):
        y = y * (1.5 - 0.5 * d * y * y)
    return y


LROW = 128
NROW = NPAD // LROW
RPT = NROW // NS
VPR = LROW // L


def _sc_body(src_hbm, dst_hbm, dinv_out, s_out,
             dst1, acc2, dinv_v, srcb, dst2, degbuf, slice_acc, idxref,
             sh_deg, sh_s, sh_dinv, sem_pf, sem_stage):
    c = lax.axis_index("c")
    t = lax.axis_index("s")
    zeros16 = jnp.zeros((L,), jnp.float32)
    ones16 = jnp.ones((L,), jnp.float32)
    iota16 = jnp.arange(L, dtype=jnp.int32)
    wid = c * NS + t

    pf_src = pltpu.async_copy(src_hbm.at[pl.ds(wid * E3, E3)], srcb, sem_pf)
    pf_dst = pltpu.async_copy(dst_hbm.at[pl.ds(wid * E3, E3)], dst2, sem_pf)

    for v in range(NROW // L):
        idxref[pl.ds(v * L, L)] = v * L + iota16

    for i in range(RPT):
        for v in range(VPR):
            degbuf[i, pl.ds(v * L, L)] = zeros16

    pltpu.sync_copy(degbuf, sh_deg.at[pl.ds(t * RPT, RPT)])
    pltpu.sync_copy(degbuf, sh_s.at[pl.ds(t * RPT, RPT)])

    pltpu.sync_copy(dst_hbm.at[pl.ds(t * E1, E1)], dst1)

    @plsc.parallel_loop(0, NROW, unroll=2)
    def _zero1(i):
        for v in range(VPR):
            acc2[i, pl.ds(v * L, L)] = zeros16

    plsc.subcore_barrier()

    @plsc.parallel_loop(0, E1 // L, unroll=8)
    def _hist(i):
        idx = dst1[pl.ds(i * L, L)]
        plsc.addupdate_scatter(acc2, [idx >> 7, idx & 127], ones16)

    pltpu.sync_copy(acc2, sh_deg.at[idxref], add=True)
    plsc.subcore_barrier()

    pltpu.sync_copy(sh_deg.at[pl.ds(t * RPT, RPT)], degbuf)

    for i in range(RPT):
        for v in range(VPR):
            slice_acc[pl.ds((i * VPR + v) * L, L)] = _fast_rsqrt(
                degbuf[i, pl.ds(v * L, L)] + 1.0)

    pltpu.sync_copy(slice_acc, sh_dinv.at[pl.ds(t * TILE_N, TILE_N)])
    pltpu.sync_copy(slice_acc, dinv_out.at[c, pl.ds(t * TILE_N, TILE_N)])
    plsc.subcore_barrier()

    dv = pltpu.async_copy(sh_dinv, dinv_v, sem_stage)

    @plsc.parallel_loop(0, NROW, unroll=2)
    def _zero2(i):
        for v in range(VPR):
            acc2[i, pl.ds(v * L, L)] = zeros16

    dv.wait()
    pf_src.wait()
    pf_dst.wait()

    @plsc.parallel_loop(0, E3 // L, unroll=8)
    def _gsc(i):
        di = dst2[pl.ds(i * L, L)]
        si = srcb[pl.ds(i * L, L)]
        g = plsc.load_gather(dinv_v, [di])
        plsc.addupdate_scatter(acc2, [si >> 7, si & 127], g)

    pltpu.sync_copy(acc2, sh_s.at[idxref], add=True)
    plsc.subcore_barrier()

    pltpu.sync_copy(sh_s.at[pl.ds(t * RPT, RPT)], degbuf)

    for i in range(RPT):
        for v in range(VPR):
            slice_acc[pl.ds((i * VPR + v) * L, L)] = degbuf[i, pl.ds(v * L, L)]

    pltpu.sync_copy(slice_acc, s_out.at[c, pl.ds(t * TILE_N, TILE_N)])


_sc_edge_kernel = functools.partial(
    pl.kernel,
    out_type=(jax.ShapeDtypeStruct((NC, NPAD), jnp.float32),
              jax.ShapeDtypeStruct((NC, NPAD), jnp.float32)),
    mesh=plsc.VectorSubcoreMesh(core_axis_name="c", subcore_axis_name="s",
                                num_cores=NC, num_subcores=NS),
    compiler_params=pltpu.CompilerParams(needs_layout_passes=False),
    scratch_types=[
        pltpu.VMEM((E1,), jnp.int32),
        pltpu.VMEM((NROW, LROW), jnp.float32),
        pltpu.VMEM((NPAD,), jnp.float32),
        pltpu.VMEM((E3,), jnp.int32),
        pltpu.VMEM((E3,), jnp.int32),
        pltpu.VMEM((RPT, LROW), jnp.float32),
        pltpu.VMEM((TILE_N,), jnp.float32),
        pltpu.VMEM((NROW,), jnp.int32),
        pltpu.VMEM_SHARED((NROW, LROW), jnp.float32),
        pltpu.VMEM_SHARED((NROW, LROW), jnp.float32),
        pltpu.VMEM_SHARED((NPAD,), jnp.float32),
        pltpu.SemaphoreType.DMA,
        pltpu.SemaphoreType.DMA,
    ],
)(_sc_body)


def _tc_body(dinv_ref, s_ref, gx_ref, wg_ref, bg_ref, xv_ref,
             w1_ref, b1_ref, w2_ref, b2_ref, wo_ref, bo_ref, out_ref):
    dinv = dinv_ref[0:1, :]
    s = s_ref[0:1, :]
    for k in range(1, NC):
        s = s + s_ref[k:k + 1, :]
    w = dinv * (s + dinv)
    wt = w[:, :N_NODES]
    dn = (((1,), (0,)), ((), ()))
    r = lax.dot_general(wt, gx_ref[...], dn, preferred_element_type=jnp.float32)
    pooled = lax.dot_general(r, wg_ref[...], dn,
                             preferred_element_type=jnp.float32)
    pooled = pooled * (1.0 / N_NODES) + bg_ref[...]
    y = jnp.concatenate([pooled, xv_ref[...]], axis=1)
    h1 = lax.dot_general(y, w1_ref[...], dn,
                         preferred_element_type=jnp.float32) + b1_ref[...]
    h1 = jnp.where(h1 >= 0, h1, 0.01 * h1)
    h2 = lax.dot_general(h1, w2_ref[...], dn,
                         preferred_element_type=jnp.float32) + b2_ref[...]
    h2 = jnp.where(h2 >= 0, h2, 0.01 * h2)
    logits = lax.dot_general(h2, wo_ref[...], dn,
                             preferred_element_type=jnp.float32) + bo_ref[...]
    m = jnp.max(logits, axis=1, keepdims=True)
    z = logits - m
    lse = jnp.log(jnp.sum(jnp.exp(z), axis=1, keepdims=True))
    out_ref[...] = z - lse


def kernel(x, gcn_x, gcn_edge_index, W_gcn, b_gcn, W1, b1, W2, b2, Wout, bout):
    ei = gcn_edge_index.astype(jnp.int32)
    dinv2, s2 = _sc_edge_kernel(ei[0], ei[1])
    out = pl.pallas_call(
        _tc_body,
        out_shape=jax.ShapeDtypeStruct((1, OUT_DIM), jnp.float32),
    )(dinv2, s2, gcn_x, W_gcn, b_gcn.reshape(1, -1), x.reshape(1, -1),
      W1, b1.reshape(1, -1), W2, b2.reshape(1, -1), Wout, bout.reshape(1, -1))
    return out.reshape(-1)


# speedup vs baseline: 1.0157x; 1.0157x over previous
import functools

import jax
import jax.numpy as jnp
from jax import lax
from jax.experimental import pallas as pl
from jax.experimental.pallas import tpu as pltpu
from jax.experimental.pallas import tpu_sc as plsc

N_NODES = 10000
OUT_DIM = 64
NPAD = 10240
E = 320000
NC = 2
NS = 16
L = 16
TILE_N = NPAD // NS
E1 = E // NS
E3 = E // (NC * NS)


def _fast_rsqrt(d):
    bi = plsc.bitcast(d, jnp.int32)
    y = plsc.bitcast(jnp.int32(0x5F3759DF) - (bi >> 1), jnp.float32)
    for _ in range(3):
        y = y * (1.5 - 0.5 * d * y * y)
    return y


LROW = 128
NROW = NPAD // LROW
RPT = NROW // NS
VPR = LROW // L


def _sc_body(src_hbm, dst_hbm, dinv_out, s_out,
             dst1, acc2, dinv_v, srcb, dst2, degbuf, slice_acc, idxref,
             sh_deg, sh_s, sh_dinv, sem_pf, sem_stage):
    c = lax.axis_index("c")
    t = lax.axis_index("s")
    zeros16 = jnp.zeros((L,), jnp.float32)
    ones16 = jnp.ones((L,), jnp.float32)
    iota16 = jnp.arange(L, dtype=jnp.int32)
    wid = c * NS + t

    pf_src = pltpu.async_copy(src_hbm.at[pl.ds(wid * E3, E3)], srcb, sem_pf)
    pf_dst = pltpu.async_copy(dst_hbm.at[pl.ds(wid * E3, E3)], dst2, sem_pf)
    ld_dst1 = pltpu.async_copy(dst_hbm.at[pl.ds(t * E1, E1)], dst1, sem_stage)

    for v in range(NROW // L):
        idxref[pl.ds(v * L, L)] = v * L + iota16

    for i in range(RPT):
        for v in range(VPR):
            degbuf[i, pl.ds(v * L, L)] = zeros16

    pltpu.sync_copy(degbuf, sh_deg.at[pl.ds(t * RPT, RPT)])
    pltpu.sync_copy(degbuf, sh_s.at[pl.ds(t * RPT, RPT)])

    @plsc.parallel_loop(0, NROW, unroll=2)
    def _zero1(i):
        for v in range(VPR):
            acc2[i, pl.ds(v * L, L)] = zeros16

    plsc.subcore_barrier()
    ld_dst1.wait()

    @plsc.parallel_loop(0, E1 // L, unroll=16)
    def _hist(i):
        idx = dst1[pl.ds(i * L, L)]
        plsc.addupdate_scatter(acc2, [idx >> 7, idx & 127], ones16)

    pltpu.sync_copy(acc2, sh_deg.at[idxref], add=True)
    plsc.subcore_barrier()

    pltpu.sync_copy(sh_deg.at[pl.ds(t * RPT, RPT)], degbuf)

    for i in range(RPT):
        for v in range(VPR):
            slice_acc[pl.ds((i * VPR + v) * L, L)] = _fast_rsqrt(
                degbuf[i, pl.ds(v * L, L)] + 1.0)

    pltpu.sync_copy(slice_acc, sh_dinv.at[pl.ds(t * TILE_N, TILE_N)])
    pltpu.sync_copy(slice_acc, dinv_out.at[c, pl.ds(t * TILE_N, TILE_N)])
    plsc.subcore_barrier()

    dv = pltpu.async_copy(sh_dinv, dinv_v, sem_stage)

    @plsc.parallel_loop(0, NROW, unroll=2)
    def _zero2(i):
        for v in range(VPR):
            acc2[i, pl.ds(v * L, L)] = zeros16

    dv.wait()
    pf_src.wait()
    pf_dst.wait()

    @plsc.parallel_loop(0, E3 // L, unroll=16)
    def _gsc(i):
        di = dst2[pl.ds(i * L, L)]
        si = srcb[pl.ds(i * L, L)]
        g = plsc.load_gather(dinv_v, [di])
        plsc.addupdate_scatter(acc2, [si >> 7, si & 127], g)

    pltpu.sync_copy(acc2, sh_s.at[idxref], add=True)
    plsc.subcore_barrier()

    pltpu.sync_copy(sh_s.at[pl.ds(t * RPT, RPT)], degbuf)

    for i in range(RPT):
        for v in range(VPR):
            slice_acc[pl.ds((i * VPR + v) * L, L)] = degbuf[i, pl.ds(v * L, L)]

    pltpu.sync_copy(slice_acc, s_out.at[c, pl.ds(t * TILE_N, TILE_N)])


_sc_edge_kernel = functools.partial(
    pl.kernel,
    out_type=(jax.ShapeDtypeStruct((NC, NPAD), jnp.float32),
              jax.ShapeDtypeStruct((NC, NPAD), jnp.float32)),
    mesh=plsc.VectorSubcoreMesh(core_axis_name="c", subcore_axis_name="s",
                                num_cores=NC, num_subcores=NS),
    compiler_params=pltpu.CompilerParams(needs_layout_passes=False),
    scratch_types=[
        pltpu.VMEM((E1,), jnp.int32),
        pltpu.VMEM((NROW, LROW), jnp.float32),
        pltpu.VMEM((NPAD,), jnp.float32),
        pltpu.VMEM((E3,), jnp.int32),
        pltpu.VMEM((E3,), jnp.int32),
        pltpu.VMEM((RPT, LROW), jnp.float32),
        pltpu.VMEM((TILE_N,), jnp.float32),
        pltpu.VMEM((NROW,), jnp.int32),
        pltpu.VMEM_SHARED((NROW, LROW), jnp.float32),
        pltpu.VMEM_SHARED((NROW, LROW), jnp.float32),
        pltpu.VMEM_SHARED((NPAD,), jnp.float32),
        pltpu.SemaphoreType.DMA,
        pltpu.SemaphoreType.DMA,
    ],
)(_sc_body)


def _tc_body(dinv_ref, s_ref, gx_ref, wg_ref, bg_ref, xv_ref,
             w1_ref, b1_ref, w2_ref, b2_ref, wo_ref, bo_ref, out_ref):
    dinv = dinv_ref[0:1, :]
    s = s_ref[0:1, :]
    for k in range(1, NC):
        s = s + s_ref[k:k + 1, :]
    w = dinv * (s + dinv)
    wt = w[:, :N_NODES]
    dn = (((1,), (0,)), ((), ()))
    r = lax.dot_general(wt, gx_ref[...], dn, preferred_element_type=jnp.float32)
    pooled = lax.dot_general(r, wg_ref[...], dn,
                             preferred_element_type=jnp.float32)
    pooled = pooled * (1.0 / N_NODES) + bg_ref[...]
    y = jnp.concatenate([pooled, xv_ref[...]], axis=1)
    h1 = lax.dot_general(y, w1_ref[...], dn,
                         preferred_element_type=jnp.float32) + b1_ref[...]
    h1 = jnp.where(h1 >= 0, h1, 0.01 * h1)
    h2 = lax.dot_general(h1, w2_ref[...], dn,
                         preferred_element_type=jnp.float32) + b2_ref[...]
    h2 = jnp.where(h2 >= 0, h2, 0.01 * h2)
    logits = lax.dot_general(h2, wo_ref[...], dn,
                             preferred_element_type=jnp.float32) + bo_ref[...]
    m = jnp.max(logits, axis=1, keepdims=True)
    z = logits - m
    lse = jnp.log(jnp.sum(jnp.exp(z), axis=1, keepdims=True))
    out_ref[...] = z - lse


def kernel(x, gcn_x, gcn_edge_index, W_gcn, b_gcn, W1, b1, W2, b2, Wout, bout):
    ei = gcn_edge_index.astype(jnp.int32)
    dinv2, s2 = _sc_edge_kernel(ei[0], ei[1])
    out = pl.pallas_call(
        _tc_body,
        out_shape=jax.ShapeDtypeStruct((1, OUT_DIM), jnp.float32),
    )(dinv2, s2, gcn_x, W_gcn, b_gcn.reshape(1, -1), x.reshape(1, -1),
      W1, b1.reshape(1, -1), W2, b2.reshape(1, -1), Wout, bout.reshape(1, -1))
    return out.reshape(-1)
